# baseline (speedup 1.0000x reference)
import functools

import jax
import jax.numpy as jnp
from jax import lax
from jax.experimental import pallas as pl
from jax.experimental.pallas import tpu as pltpu
from jax.experimental.pallas import tpu_sc as plsc

N_NODES = 10000
C = 128
K = 32
LANES = 16

N_SC = 2
N_TILES = 16
FPT = C // N_TILES
N_PAD = 10240
NODES_PER_SC = N_PAD // N_SC
CHUNK_N = 128
SC_CHUNKS = NODES_PER_SC // CHUNK_N
GROUPS = CHUNK_N // LANES

MM_BLOCK = 1280


def _mm_body(x_ref, w_ref, bias_ref, ei_ref, a_ref, bp_ref, eit_ref):
    i = pl.program_id(0)
    rows = i * MM_BLOCK + lax.broadcasted_iota(jnp.int32, (MM_BLOCK, K), 0)
    eit_ref[...] = jnp.where(rows < N_NODES, ei_ref[...], 0).T
    xb = x_ref[...].astype(jnp.bfloat16)
    w1 = w_ref[:, :C]
    w2 = w_ref[:, C:]
    w2b = w2.astype(jnp.bfloat16)
    dn = (((1,), (1,)), ((), ()))
    a_ref[...] = (lax.dot_general(xb, (w1 - w2).astype(jnp.bfloat16), dn,
                                  preferred_element_type=jnp.float32)
                  + bias_ref[...]).astype(jnp.bfloat16)
    btf = lax.dot_general(w2b, xb, dn, preferred_element_type=jnp.float32)
    bb = btf.astype(jnp.bfloat16)
    lo = lax.bitcast_convert_type(bb[:C // 2, :], jnp.uint16).astype(jnp.uint32)
    hi = lax.bitcast_convert_type(bb[C // 2:, :], jnp.uint16).astype(jnp.uint32)
    bp_ref[...] = lax.bitcast_convert_type(lo | (hi << 16), jnp.int32)


@jax.jit
def _mm_call(x, W, bias, ei):
    grid = (N_PAD // MM_BLOCK,)
    return pl.pallas_call(
        _mm_body,
        grid=grid,
        in_specs=[
            pl.BlockSpec((MM_BLOCK, C), lambda i: (i, 0)),
            pl.BlockSpec((C, 2 * C), lambda i: (0, 0)),
            pl.BlockSpec((1, C), lambda i: (0, 0)),
            pl.BlockSpec((MM_BLOCK, K), lambda i: (i, 0)),
        ],
        out_specs=[
            pl.BlockSpec((MM_BLOCK, C), lambda i: (i, 0)),
            pl.BlockSpec((C // 2, MM_BLOCK), lambda i: (0, i)),
            pl.BlockSpec((K, MM_BLOCK), lambda i: (0, i)),
        ],
        out_shape=[
            jax.ShapeDtypeStruct((N_PAD, C), jnp.bfloat16),
            jax.ShapeDtypeStruct((C // 2, N_PAD), jnp.int32),
            jax.ShapeDtypeStruct((K, N_PAD), jnp.int32),
        ],
    )(x, W, bias, ei)


def _elu_body(a_ref, mt_ref, o_ref):
    w = lax.bitcast_convert_type(mt_ref[...], jnp.uint32)
    lo = lax.bitcast_convert_type(
        (w & 0xFFFF).astype(jnp.uint16), jnp.bfloat16).astype(jnp.float32)
    hi = lax.bitcast_convert_type(
        (w >> 16).astype(jnp.uint16), jnp.bfloat16).astype(jnp.float32)
    m = jnp.concatenate([lo, hi], axis=0)
    z = a_ref[...].astype(jnp.float32) + m.T
    o_ref[...] = jnp.where(z > 0, z, jnp.exp(z) - 1.0)


@jax.jit
def _elu_call(A, Mt):
    grid = (N_PAD // MM_BLOCK,)
    return pl.pallas_call(
        _elu_body,
        grid=grid,
        in_specs=[
            pl.BlockSpec((MM_BLOCK, C), lambda i: (i, 0)),
            pl.BlockSpec((C // 2, MM_BLOCK), lambda i: (0, i)),
        ],
        out_specs=pl.BlockSpec((MM_BLOCK, C), lambda i: (i, 0)),
        out_shape=jax.ShapeDtypeStruct((N_NODES, C), jnp.float32),
    )(A, Mt)


def _sc_body(bt_hbm, idx_hbm, mt_hbm, b_v,
             idx0, idx1, o0, o1, isem0, isem1, osem0, osem1):
    sc = lax.axis_index("c")
    tile = lax.axis_index("s")
    nbase = sc * NODES_PER_SC
    pltpu.sync_copy(bt_hbm.at[pl.ds(tile * (FPT // 2), FPT // 2)], b_v)

    idx_bufs = (idx0, idx1)
    idx_sems = (isem0, isem1)
    o_bufs = (o0, o1)
    o_sems = (osem0, osem1)

    def start_idx(ch, buf, sm):
        pltpu.make_async_copy(
            idx_hbm.at[:, pl.ds(nbase + ch * CHUNK_N, CHUNK_N)],
            buf, sm).start()

    def wait_idx(buf, sm):
        pltpu.make_async_copy(
            idx_hbm.at[:, pl.ds(0, CHUNK_N)], buf, sm).wait()

    def wait_store(buf, sm):
        pltpu.make_async_copy(
            buf, mt_hbm.at[pl.ds(0, FPT // 2), pl.ds(0, CHUNK_N)], sm).wait()

    iota = lax.iota(jnp.int32, LANES)

    def compute_chunk(idx_v, o_v):
        def group_body(g, carry):
            accs = [None] * (FPT // 2)
            for k in range(K):
                nid = idx_v[k, pl.ds(g * LANES, LANES)]
                for fp in range(FPT // 2):
                    w = plsc.load_gather(
                        b_v, [jnp.full((LANES,), fp, jnp.int32), nid])
                    v = plsc.bitcast(w, jnp.bfloat16)
                    accs[fp] = v if k == 0 else jnp.maximum(accs[fp], v)
            for fp in range(FPT // 2):
                o_v[fp, pl.ds(g * LANES, LANES)] = plsc.bitcast(
                    accs[fp], jnp.int32)
            return carry
        lax.fori_loop(0, GROUPS, group_body, 0)

    start_idx(0, idx_bufs[0], idx_sems[0])

    def pair_body(i, carry):
        base = i * 2
        for par in range(2):
            ch = base + par
            @pl.when(ch + 1 < SC_CHUNKS)
            def _():
                start_idx(ch + 1, idx_bufs[1 - par], idx_sems[1 - par])
            wait_idx(idx_bufs[par], idx_sems[par])
            @pl.when(ch >= 2)
            def _():
                wait_store(o_bufs[par], o_sems[par])
            compute_chunk(idx_bufs[par], o_bufs[par])
            pltpu.make_async_copy(
                o_bufs[par],
                mt_hbm.at[pl.ds(tile * (FPT // 2), FPT // 2),
                          pl.ds(nbase + ch * CHUNK_N, CHUNK_N)],
                o_sems[par]).start()
        return carry

    lax.fori_loop(0, SC_CHUNKS // 2, pair_body, 0)
    wait_store(o_bufs[0], o_sems[0])
    wait_store(o_bufs[1], o_sems[1])


@jax.jit
def _sc_call(Bt, idx):
    mesh = plsc.VectorSubcoreMesh(core_axis_name="c", subcore_axis_name="s")
    f = functools.partial(
        pl.kernel,
        out_type=jax.ShapeDtypeStruct((C // 2, N_PAD), jnp.int32),
        mesh=mesh,
        compiler_params=pltpu.CompilerParams(needs_layout_passes=False),
        scratch_types=(
            [pltpu.VMEM((FPT // 2, N_PAD), jnp.int32)]
            + [pltpu.VMEM((K, CHUNK_N), jnp.int32)] * 2
            + [pltpu.VMEM((FPT // 2, CHUNK_N), jnp.int32)] * 2
            + [pltpu.SemaphoreType.DMA] * 4
        ),
    )(_sc_body)
    return f(Bt, idx)


def kernel(x, edge_index, W, b):
    x = x.astype(jnp.float32)
    ei = edge_index.astype(jnp.int32)
    A, Bp, ei_t = _mm_call(x, W, b.reshape(1, C), ei)
    Mt = _sc_call(Bp, ei_t)
    return _elu_call(A, Mt)


# speedup vs baseline: 1.0445x; 1.0445x over previous
import functools

import jax
import jax.numpy as jnp
from jax import lax
from jax.experimental import pallas as pl
from jax.experimental.pallas import tpu as pltpu
from jax.experimental.pallas import tpu_sc as plsc

N_NODES = 10000
C = 128
K = 32
LANES = 16

N_SC = 2
N_TILES = 16
FPT = C // N_TILES
N_PAD = 10240
NODES_PER_SC = N_PAD // N_SC
CHUNK_N = 128
SC_CHUNKS = NODES_PER_SC // CHUNK_N
GROUPS = CHUNK_N // LANES

MM_BLOCK = 1280


def _mm_body(x_ref, w_ref, bias_ref, a_ref, bp_ref):
    xb = x_ref[...]
    w1 = w_ref[:, :C]
    w2 = w_ref[:, C:]
    dn = (((1,), (1,)), ((), ()))
    a_ref[...] = (lax.dot_general(xb, w1 - w2, dn,
                                  preferred_element_type=jnp.float32)
                  + bias_ref[...]).astype(jnp.bfloat16)
    btf = lax.dot_general(w2, xb, dn, preferred_element_type=jnp.float32)
    bb = btf.astype(jnp.bfloat16)
    lo = lax.bitcast_convert_type(bb[:C // 2, :], jnp.uint16).astype(jnp.uint32)
    hi = lax.bitcast_convert_type(bb[C // 2:, :], jnp.uint16).astype(jnp.uint32)
    bp_ref[...] = lax.bitcast_convert_type(lo | (hi << 16), jnp.int32)


@jax.jit
def _mm_call(x, W, bias):
    grid = (N_PAD // MM_BLOCK,)
    return pl.pallas_call(
        _mm_body,
        grid=grid,
        in_specs=[
            pl.BlockSpec((MM_BLOCK, C), lambda i: (i, 0)),
            pl.BlockSpec((C, 2 * C), lambda i: (0, 0)),
            pl.BlockSpec((1, C), lambda i: (0, 0)),
        ],
        out_specs=[
            pl.BlockSpec((MM_BLOCK, C), lambda i: (i, 0)),
            pl.BlockSpec((C // 2, MM_BLOCK), lambda i: (0, i)),
        ],
        out_shape=[
            jax.ShapeDtypeStruct((N_PAD, C), jnp.bfloat16),
            jax.ShapeDtypeStruct((C // 2, N_PAD), jnp.int32),
        ],
    )(x, W, bias)


def _elu_body(a_ref, mt_ref, o_ref):
    w = lax.bitcast_convert_type(mt_ref[...], jnp.uint32)
    lo = lax.bitcast_convert_type(
        (w & 0xFFFF).astype(jnp.uint16), jnp.bfloat16).astype(jnp.float32)
    hi = lax.bitcast_convert_type(
        (w >> 16).astype(jnp.uint16), jnp.bfloat16).astype(jnp.float32)
    m = jnp.concatenate([lo, hi], axis=0)
    z = a_ref[...].astype(jnp.float32) + m.T
    o_ref[...] = jnp.where(z > 0, z, jnp.exp(z) - 1.0)


@jax.jit
def _elu_call(A, Mt):
    grid = (N_PAD // MM_BLOCK,)
    return pl.pallas_call(
        _elu_body,
        grid=grid,
        in_specs=[
            pl.BlockSpec((MM_BLOCK, C), lambda i: (i, 0)),
            pl.BlockSpec((C // 2, MM_BLOCK), lambda i: (0, i)),
        ],
        out_specs=pl.BlockSpec((MM_BLOCK, C), lambda i: (i, 0)),
        out_shape=jax.ShapeDtypeStruct((N_NODES, C), jnp.float32),
    )(A, Mt)


def _sc_body(bt_hbm, idx_hbm, mt_hbm, b_v,
             idx0, idx1, o0, o1, isem0, isem1, osem0, osem1):
    sc = lax.axis_index("c")
    tile = lax.axis_index("s")
    nbase = sc * NODES_PER_SC
    pltpu.sync_copy(bt_hbm.at[pl.ds(tile * (FPT // 2), FPT // 2)], b_v)

    idx_bufs = (idx0, idx1)
    idx_sems = (isem0, isem1)
    o_bufs = (o0, o1)
    o_sems = (osem0, osem1)

    def start_idx(ch, buf, sm):
        pltpu.make_async_copy(
            idx_hbm.at[:, pl.ds(nbase + ch * CHUNK_N, CHUNK_N)],
            buf, sm).start()

    def wait_idx(buf, sm):
        pltpu.make_async_copy(
            idx_hbm.at[:, pl.ds(0, CHUNK_N)], buf, sm).wait()

    def wait_store(buf, sm):
        pltpu.make_async_copy(
            buf, mt_hbm.at[pl.ds(0, FPT // 2), pl.ds(0, CHUNK_N)], sm).wait()

    iota = lax.iota(jnp.int32, LANES)

    def compute_chunk(idx_v, o_v):
        def group_body(g, carry):
            accs = [None] * (FPT // 2)
            for k in range(K):
                nid = idx_v[k, pl.ds(g * LANES, LANES)]
                for fp in range(FPT // 2):
                    w = plsc.load_gather(
                        b_v, [jnp.full((LANES,), fp, jnp.int32), nid])
                    v = plsc.bitcast(w, jnp.bfloat16)
                    accs[fp] = v if k == 0 else jnp.maximum(accs[fp], v)
            for fp in range(FPT // 2):
                o_v[fp, pl.ds(g * LANES, LANES)] = plsc.bitcast(
                    accs[fp], jnp.int32)
            return carry
        lax.fori_loop(0, GROUPS, group_body, 0)

    start_idx(0, idx_bufs[0], idx_sems[0])

    def pair_body(i, carry):
        base = i * 2
        for par in range(2):
            ch = base + par
            @pl.when(ch + 1 < SC_CHUNKS)
            def _():
                start_idx(ch + 1, idx_bufs[1 - par], idx_sems[1 - par])
            wait_idx(idx_bufs[par], idx_sems[par])
            @pl.when(ch >= 2)
            def _():
                wait_store(o_bufs[par], o_sems[par])
            compute_chunk(idx_bufs[par], o_bufs[par])
            pltpu.make_async_copy(
                o_bufs[par],
                mt_hbm.at[pl.ds(tile * (FPT // 2), FPT // 2),
                          pl.ds(nbase + ch * CHUNK_N, CHUNK_N)],
                o_sems[par]).start()
        return carry

    lax.fori_loop(0, SC_CHUNKS // 2, pair_body, 0)
    wait_store(o_bufs[0], o_sems[0])
    wait_store(o_bufs[1], o_sems[1])


@jax.jit
def _sc_call(Bt, idx):
    mesh = plsc.VectorSubcoreMesh(core_axis_name="c", subcore_axis_name="s")
    f = functools.partial(
        pl.kernel,
        out_type=jax.ShapeDtypeStruct((C // 2, N_PAD), jnp.int32),
        mesh=mesh,
        compiler_params=pltpu.CompilerParams(needs_layout_passes=False),
        scratch_types=(
            [pltpu.VMEM((FPT // 2, N_PAD), jnp.int32)]
            + [pltpu.VMEM((K, CHUNK_N), jnp.int32)] * 2
            + [pltpu.VMEM((FPT // 2, CHUNK_N), jnp.int32)] * 2
            + [pltpu.SemaphoreType.DMA] * 4
        ),
    )(_sc_body)
    return f(Bt, idx)


def kernel(x, edge_index, W, b):
    x = x.astype(jnp.float32)
    ei = edge_index.astype(jnp.int32)
    A, Bp = _mm_call(x, W, b.reshape(1, C))
    ei_pad = jnp.concatenate(
        [ei, jnp.zeros((N_PAD - N_NODES, K), jnp.int32)], axis=0)
    Mt = _sc_call(Bp, ei_pad.T)
    return _elu_call(A, Mt)


# speedup vs baseline: 1.0527x; 1.0079x over previous
import functools

import jax
import jax.numpy as jnp
from jax import lax
from jax.experimental import pallas as pl
from jax.experimental.pallas import tpu as pltpu
from jax.experimental.pallas import tpu_sc as plsc

N_NODES = 10000
C = 128
K = 32
LANES = 16

N_SC = 2
N_TILES = 16
FPT = C // N_TILES
N_PAD = 10240
NODES_PER_SC = N_PAD // N_SC
CHUNK_N = 256
SC_CHUNKS = NODES_PER_SC // CHUNK_N
GROUPS = CHUNK_N // LANES

MM_BLOCK = 1280


def _mm_body(x_ref, w_ref, bias_ref, a_ref, bp_ref):
    xb = x_ref[...]
    w1 = w_ref[:, :C]
    w2 = w_ref[:, C:]
    dn = (((1,), (1,)), ((), ()))
    a_ref[...] = (lax.dot_general(xb, w1 - w2, dn,
                                  preferred_element_type=jnp.float32)
                  + bias_ref[...]).astype(jnp.bfloat16)
    btf = lax.dot_general(w2, xb, dn, preferred_element_type=jnp.float32)
    bb = btf.astype(jnp.bfloat16)
    lo = lax.bitcast_convert_type(bb[:C // 2, :], jnp.uint16).astype(jnp.uint32)
    hi = lax.bitcast_convert_type(bb[C // 2:, :], jnp.uint16).astype(jnp.uint32)
    bp_ref[...] = lax.bitcast_convert_type(lo | (hi << 16), jnp.int32)


@jax.jit
def _mm_call(x, W, bias):
    grid = (N_PAD // MM_BLOCK,)
    return pl.pallas_call(
        _mm_body,
        grid=grid,
        in_specs=[
            pl.BlockSpec((MM_BLOCK, C), lambda i: (i, 0)),
            pl.BlockSpec((C, 2 * C), lambda i: (0, 0)),
            pl.BlockSpec((1, C), lambda i: (0, 0)),
        ],
        out_specs=[
            pl.BlockSpec((MM_BLOCK, C), lambda i: (i, 0)),
            pl.BlockSpec((C // 2, MM_BLOCK), lambda i: (0, i)),
        ],
        out_shape=[
            jax.ShapeDtypeStruct((N_PAD, C), jnp.bfloat16),
            jax.ShapeDtypeStruct((C // 2, N_PAD), jnp.int32),
        ],
    )(x, W, bias)


def _elu_body(a_ref, mt_ref, o_ref):
    w = lax.bitcast_convert_type(mt_ref[...], jnp.uint32)
    lo = lax.bitcast_convert_type(
        (w & 0xFFFF).astype(jnp.uint16), jnp.bfloat16).astype(jnp.float32)
    hi = lax.bitcast_convert_type(
        (w >> 16).astype(jnp.uint16), jnp.bfloat16).astype(jnp.float32)
    m = jnp.concatenate([lo, hi], axis=0)
    z = a_ref[...].astype(jnp.float32) + m.T
    o_ref[...] = jnp.where(z > 0, z, jnp.exp(z) - 1.0)


@jax.jit
def _elu_call(A, Mt):
    grid = (N_PAD // MM_BLOCK,)
    return pl.pallas_call(
        _elu_body,
        grid=grid,
        in_specs=[
            pl.BlockSpec((MM_BLOCK, C), lambda i: (i, 0)),
            pl.BlockSpec((C // 2, MM_BLOCK), lambda i: (0, i)),
        ],
        out_specs=pl.BlockSpec((MM_BLOCK, C), lambda i: (i, 0)),
        out_shape=jax.ShapeDtypeStruct((N_NODES, C), jnp.float32),
    )(A, Mt)


def _sc_body(bt_hbm, idx_hbm, mt_hbm, b_v,
             idx0, idx1, o0, o1, isem0, isem1, osem0, osem1):
    sc = lax.axis_index("c")
    tile = lax.axis_index("s")
    nbase = sc * NODES_PER_SC
    idx_bufs = (idx0, idx1)
    idx_sems = (isem0, isem1)
    o_bufs = (o0, o1)
    o_sems = (osem0, osem1)

    def start_idx(ch, buf, sm):
        pltpu.make_async_copy(
            idx_hbm.at[:, pl.ds(nbase + ch * CHUNK_N, CHUNK_N)],
            buf, sm).start()

    def wait_idx(buf, sm):
        pltpu.make_async_copy(
            idx_hbm.at[:, pl.ds(0, CHUNK_N)], buf, sm).wait()

    def wait_store(buf, sm):
        pltpu.make_async_copy(
            buf, mt_hbm.at[pl.ds(0, FPT // 2), pl.ds(0, CHUNK_N)], sm).wait()

    iota = lax.iota(jnp.int32, LANES)

    def compute_chunk(idx_v, o_v):
        def group_body(g, carry):
            accs = [None] * (FPT // 2)
            for k in range(K):
                nid = idx_v[k, pl.ds(g * LANES, LANES)]
                for fp in range(FPT // 2):
                    w = plsc.load_gather(
                        b_v, [jnp.full((LANES,), fp, jnp.int32), nid])
                    v = plsc.bitcast(w, jnp.bfloat16)
                    accs[fp] = v if k == 0 else jnp.maximum(accs[fp], v)
            for fp in range(FPT // 2):
                o_v[fp, pl.ds(g * LANES, LANES)] = plsc.bitcast(
                    accs[fp], jnp.int32)
            return carry
        lax.fori_loop(0, GROUPS, group_body, 0)

    stage = pltpu.make_async_copy(
        bt_hbm.at[pl.ds(tile * (FPT // 2), FPT // 2)], b_v, isem1)
    stage.start()
    start_idx(0, idx_bufs[0], idx_sems[0])
    stage.wait()

    def pair_body(i, carry):
        base = i * 2
        for par in range(2):
            ch = base + par
            @pl.when(ch + 1 < SC_CHUNKS)
            def _():
                start_idx(ch + 1, idx_bufs[1 - par], idx_sems[1 - par])
            wait_idx(idx_bufs[par], idx_sems[par])
            @pl.when(ch >= 2)
            def _():
                wait_store(o_bufs[par], o_sems[par])
            compute_chunk(idx_bufs[par], o_bufs[par])
            pltpu.make_async_copy(
                o_bufs[par],
                mt_hbm.at[pl.ds(tile * (FPT // 2), FPT // 2),
                          pl.ds(nbase + ch * CHUNK_N, CHUNK_N)],
                o_sems[par]).start()
        return carry

    lax.fori_loop(0, SC_CHUNKS // 2, pair_body, 0)
    wait_store(o_bufs[0], o_sems[0])
    wait_store(o_bufs[1], o_sems[1])


@jax.jit
def _sc_call(Bt, idx):
    mesh = plsc.VectorSubcoreMesh(core_axis_name="c", subcore_axis_name="s")
    f = functools.partial(
        pl.kernel,
        out_type=jax.ShapeDtypeStruct((C // 2, N_PAD), jnp.int32),
        mesh=mesh,
        compiler_params=pltpu.CompilerParams(needs_layout_passes=False),
        scratch_types=(
            [pltpu.VMEM((FPT // 2, N_PAD), jnp.int32)]
            + [pltpu.VMEM((K, CHUNK_N), jnp.int32)] * 2
            + [pltpu.VMEM((FPT // 2, CHUNK_N), jnp.int32)] * 2
            + [pltpu.SemaphoreType.DMA] * 4
        ),
    )(_sc_body)
    return f(Bt, idx)


def kernel(x, edge_index, W, b):
    x = x.astype(jnp.float32)
    ei = edge_index.astype(jnp.int32)
    A, Bp = _mm_call(x, W, b.reshape(1, C))
    ei_pad = jnp.concatenate(
        [ei, jnp.zeros((N_PAD - N_NODES, K), jnp.int32)], axis=0)
    Mt = _sc_call(Bp, ei_pad.T)
    return _elu_call(A, Mt)


# speedup vs baseline: 1.0532x; 1.0005x over previous
import functools

import jax
import jax.numpy as jnp
from jax import lax
from jax.experimental import pallas as pl
from jax.experimental.pallas import tpu as pltpu
from jax.experimental.pallas import tpu_sc as plsc

N_NODES = 10000
C = 128
K = 32
LANES = 16

N_SC = 2
N_TILES = 16
FPT = C // N_TILES
N_PAD = 10240
NODES_PER_SC = N_PAD // N_SC
CHUNK_N = 256
SC_CHUNKS = NODES_PER_SC // CHUNK_N
GROUPS = CHUNK_N // LANES

MM_BLOCK = 1280


def _mm_body(x_ref, w_ref, bias_ref, a_ref, bp_ref):
    xb = x_ref[...]
    w1 = w_ref[:, :C]
    w2 = w_ref[:, C:]
    dn = (((1,), (1,)), ((), ()))
    a_ref[...] = (lax.dot_general(xb, w1 - w2, dn,
                                  preferred_element_type=jnp.float32)
                  + bias_ref[...]).astype(jnp.bfloat16)
    btf = lax.dot_general(w2, xb, dn, preferred_element_type=jnp.float32)
    bb = btf.astype(jnp.bfloat16)
    lo = lax.bitcast_convert_type(bb[:C // 2, :], jnp.uint16).astype(jnp.uint32)
    hi = lax.bitcast_convert_type(bb[C // 2:, :], jnp.uint16).astype(jnp.uint32)
    bp_ref[...] = lax.bitcast_convert_type(lo | (hi << 16), jnp.int32)


@jax.jit
def _mm_call(x, W, bias):
    grid = (N_PAD // MM_BLOCK,)
    return pl.pallas_call(
        _mm_body,
        grid=grid,
        in_specs=[
            pl.BlockSpec((MM_BLOCK, C), lambda i: (i, 0)),
            pl.BlockSpec((C, 2 * C), lambda i: (0, 0)),
            pl.BlockSpec((1, C), lambda i: (0, 0)),
        ],
        out_specs=[
            pl.BlockSpec((MM_BLOCK, C), lambda i: (i, 0)),
            pl.BlockSpec((C // 2, MM_BLOCK), lambda i: (0, i)),
        ],
        out_shape=[
            jax.ShapeDtypeStruct((N_PAD, C), jnp.bfloat16),
            jax.ShapeDtypeStruct((C // 2, N_PAD), jnp.int32),
        ],
    )(x, W, bias)


def _elu_body(a_ref, mt_ref, o_ref):
    w = lax.bitcast_convert_type(mt_ref[...], jnp.uint32)
    lo = lax.bitcast_convert_type(
        (w & 0xFFFF).astype(jnp.uint16), jnp.bfloat16).astype(jnp.float32)
    hi = lax.bitcast_convert_type(
        (w >> 16).astype(jnp.uint16), jnp.bfloat16).astype(jnp.float32)
    m = jnp.concatenate([lo, hi], axis=0)
    z = a_ref[...].astype(jnp.float32) + m.T
    o_ref[...] = jnp.where(z > 0, z, jnp.exp(z) - 1.0)


@jax.jit
def _elu_call(A, Mt):
    grid = (N_PAD // MM_BLOCK,)
    return pl.pallas_call(
        _elu_body,
        grid=grid,
        in_specs=[
            pl.BlockSpec((MM_BLOCK, C), lambda i: (i, 0)),
            pl.BlockSpec((C // 2, MM_BLOCK), lambda i: (0, i)),
        ],
        out_specs=pl.BlockSpec((MM_BLOCK, C), lambda i: (i, 0)),
        out_shape=jax.ShapeDtypeStruct((N_NODES, C), jnp.float32),
    )(A, Mt)


def _sc_body(bt_hbm, idx_hbm, mt_hbm, b_v,
             idx0, idx1, o0, o1, isem0, isem1, osem0, osem1):
    sc = lax.axis_index("c")
    tile = lax.axis_index("s")
    nbase = sc * NODES_PER_SC
    idx_bufs = (idx0, idx1)
    idx_sems = (isem0, isem1)
    o_bufs = (o0, o1)
    o_sems = (osem0, osem1)

    def start_idx(ch, buf, sm):
        pltpu.make_async_copy(
            idx_hbm.at[:, pl.ds(nbase + ch * CHUNK_N, CHUNK_N)],
            buf, sm).start()

    def wait_idx(buf, sm):
        pltpu.make_async_copy(
            idx_hbm.at[:, pl.ds(0, CHUNK_N)], buf, sm).wait()

    def wait_store(buf, sm):
        pltpu.make_async_copy(
            buf, mt_hbm.at[pl.ds(0, FPT // 2), pl.ds(0, CHUNK_N)], sm).wait()

    def compute_chunk(idx_v, o_v):
        def group_body(g, carry):
            accs = [None] * (FPT // 2)
            for k in range(K):
                nid = idx_v[k, pl.ds(g * LANES, LANES)]
                for fp in range(FPT // 2):
                    w = plsc.load_gather(
                        b_v, [jnp.full((LANES,), fp, jnp.int32), nid])
                    v = plsc.bitcast(w, jnp.bfloat16)
                    accs[fp] = v if k == 0 else jnp.maximum(accs[fp], v)
            for fp in range(FPT // 2):
                o_v[fp, pl.ds(g * LANES, LANES)] = plsc.bitcast(
                    accs[fp], jnp.int32)
            return carry
        lax.fori_loop(0, GROUPS, group_body, 0)

    stage = pltpu.make_async_copy(
        bt_hbm.at[pl.ds(tile * (FPT // 2), FPT // 2)], b_v, isem1)
    stage.start()
    start_idx(0, idx_bufs[0], idx_sems[0])
    stage.wait()

    def pair_body(i, carry):
        base = i * 2
        for par in range(2):
            ch = base + par
            @pl.when(ch + 1 < SC_CHUNKS)
            def _():
                start_idx(ch + 1, idx_bufs[1 - par], idx_sems[1 - par])
            wait_idx(idx_bufs[par], idx_sems[par])
            @pl.when(ch >= 2)
            def _():
                wait_store(o_bufs[par], o_sems[par])
            compute_chunk(idx_bufs[par], o_bufs[par])
            pltpu.make_async_copy(
                o_bufs[par],
                mt_hbm.at[pl.ds(tile * (FPT // 2), FPT // 2),
                          pl.ds(nbase + ch * CHUNK_N, CHUNK_N)],
                o_sems[par]).start()
        return carry

    lax.fori_loop(0, SC_CHUNKS // 2, pair_body, 0)
    wait_store(o_bufs[0], o_sems[0])
    wait_store(o_bufs[1], o_sems[1])


@jax.jit
def _sc_call(Bt, idx):
    mesh = plsc.VectorSubcoreMesh(core_axis_name="c", subcore_axis_name="s")
    f = functools.partial(
        pl.kernel,
        out_type=jax.ShapeDtypeStruct((C // 2, N_PAD), jnp.int32),
        mesh=mesh,
        compiler_params=pltpu.CompilerParams(needs_layout_passes=False),
        scratch_types=(
            [pltpu.VMEM((FPT // 2, N_PAD), jnp.int32)]
            + [pltpu.VMEM((K, CHUNK_N), jnp.int32)] * 2
            + [pltpu.VMEM((FPT // 2, CHUNK_N), jnp.int32)] * 2
            + [pltpu.SemaphoreType.DMA] * 4
        ),
    )(_sc_body)
    return f(Bt, idx)


def kernel(x, edge_index, W, b):
    x = x.astype(jnp.float32)
    ei = edge_index.astype(jnp.int32)
    A, Bp = _mm_call(x, W, b.reshape(1, C))
    ei_pad = jnp.concatenate(
        [ei, jnp.zeros((N_PAD - N_NODES, K), jnp.int32)], axis=0)
    Mt = _sc_call(Bp, ei_pad.T)
    return _elu_call(A, Mt)


# speedup vs baseline: 1.0928x; 1.0375x over previous
import functools

import jax
import jax.numpy as jnp
from jax import lax
from jax.experimental import pallas as pl
from jax.experimental.pallas import tpu as pltpu
from jax.experimental.pallas import tpu_sc as plsc

N_NODES = 10000
C = 128
K = 32
LANES = 16

N_SC = 2
N_TILES = 16
FPT = C // N_TILES
N_PAD = 10240
NODES_PER_SC = N_PAD // N_SC
CHUNK_N = 512
SC_CHUNKS = NODES_PER_SC // CHUNK_N
GROUPS = CHUNK_N // LANES

MM_BLOCK = 1280


def _mm_body(x_ref, w_ref, bias_ref, a_ref, bp_ref):
    xb = x_ref[...]
    w1 = w_ref[:, :C]
    w2 = w_ref[:, C:]
    dn = (((1,), (1,)), ((), ()))
    a_ref[...] = (lax.dot_general(xb, w1 - w2, dn,
                                  preferred_element_type=jnp.float32)
                  + bias_ref[...]).astype(jnp.bfloat16)
    btf = lax.dot_general(w2, xb, dn, preferred_element_type=jnp.float32)
    bb = btf.astype(jnp.bfloat16)
    lo = lax.bitcast_convert_type(bb[:C // 2, :], jnp.uint16).astype(jnp.uint32)
    hi = lax.bitcast_convert_type(bb[C // 2:, :], jnp.uint16).astype(jnp.uint32)
    bp_ref[...] = lax.bitcast_convert_type(lo | (hi << 16), jnp.int32)


@jax.jit
def _mm_call(x, W, bias):
    grid = (N_PAD // MM_BLOCK,)
    return pl.pallas_call(
        _mm_body,
        grid=grid,
        in_specs=[
            pl.BlockSpec((MM_BLOCK, C), lambda i: (i, 0)),
            pl.BlockSpec((C, 2 * C), lambda i: (0, 0)),
            pl.BlockSpec((1, C), lambda i: (0, 0)),
        ],
        out_specs=[
            pl.BlockSpec((MM_BLOCK, C), lambda i: (i, 0)),
            pl.BlockSpec((C // 2, MM_BLOCK), lambda i: (0, i)),
        ],
        out_shape=[
            jax.ShapeDtypeStruct((N_PAD, C), jnp.bfloat16),
            jax.ShapeDtypeStruct((C // 2, N_PAD), jnp.int32),
        ],
    )(x, W, bias)


def _elu_body(a_ref, mt_ref, o_ref):
    w = lax.bitcast_convert_type(mt_ref[...], jnp.uint32)
    lo = lax.bitcast_convert_type(
        (w & 0xFFFF).astype(jnp.uint16), jnp.bfloat16).astype(jnp.float32)
    hi = lax.bitcast_convert_type(
        (w >> 16).astype(jnp.uint16), jnp.bfloat16).astype(jnp.float32)
    m = jnp.concatenate([lo, hi], axis=0)
    z = a_ref[...].astype(jnp.float32) + m.T
    o_ref[...] = jnp.where(z > 0, z, jnp.exp(z) - 1.0)


@jax.jit
def _elu_call(A, Mt):
    grid = (N_PAD // MM_BLOCK,)
    return pl.pallas_call(
        _elu_body,
        grid=grid,
        in_specs=[
            pl.BlockSpec((MM_BLOCK, C), lambda i: (i, 0)),
            pl.BlockSpec((C // 2, MM_BLOCK), lambda i: (0, i)),
        ],
        out_specs=pl.BlockSpec((MM_BLOCK, C), lambda i: (i, 0)),
        out_shape=jax.ShapeDtypeStruct((N_NODES, C), jnp.float32),
    )(A, Mt)


def _sc_body(bt_hbm, idx_hbm, mt_hbm, b_v,
             idx0, idx1, o0, o1, isem0, isem1, osem0, osem1):
    sc = lax.axis_index("c")
    tile = lax.axis_index("s")
    nbase = sc * NODES_PER_SC
    idx_bufs = (idx0, idx1)
    idx_sems = (isem0, isem1)
    o_bufs = (o0, o1)
    o_sems = (osem0, osem1)

    def start_idx(ch, buf, sm):
        pltpu.make_async_copy(
            idx_hbm.at[:, pl.ds(nbase + ch * CHUNK_N, CHUNK_N)],
            buf, sm).start()

    def wait_idx(buf, sm):
        pltpu.make_async_copy(
            idx_hbm.at[:, pl.ds(0, CHUNK_N)], buf, sm).wait()

    def wait_store(buf, sm):
        pltpu.make_async_copy(
            buf, mt_hbm.at[pl.ds(0, FPT // 2), pl.ds(0, CHUNK_N)], sm).wait()

    def compute_chunk(idx_v, o_v):
        def group_body(g, carry):
            accs = [None] * (FPT // 2)
            for k in range(K):
                nid = idx_v[k, pl.ds(g * LANES, LANES)]
                for fp in range(FPT // 2):
                    w = plsc.load_gather(
                        b_v, [jnp.full((LANES,), fp, jnp.int32), nid])
                    v = plsc.bitcast(w, jnp.bfloat16)
                    accs[fp] = v if k == 0 else jnp.maximum(accs[fp], v)
            for fp in range(FPT // 2):
                o_v[fp, pl.ds(g * LANES, LANES)] = plsc.bitcast(
                    accs[fp], jnp.int32)
            return carry
        lax.fori_loop(0, GROUPS, group_body, 0)

    stage = pltpu.make_async_copy(
        bt_hbm.at[pl.ds(tile * (FPT // 2), FPT // 2)], b_v, isem1)
    stage.start()
    start_idx(0, idx_bufs[0], idx_sems[0])
    stage.wait()

    def pair_body(i, carry):
        base = i * 2
        for par in range(2):
            ch = base + par
            @pl.when(ch + 1 < SC_CHUNKS)
            def _():
                start_idx(ch + 1, idx_bufs[1 - par], idx_sems[1 - par])
            wait_idx(idx_bufs[par], idx_sems[par])
            @pl.when(ch >= 2)
            def _():
                wait_store(o_bufs[par], o_sems[par])
            compute_chunk(idx_bufs[par], o_bufs[par])
            pltpu.make_async_copy(
                o_bufs[par],
                mt_hbm.at[pl.ds(tile * (FPT // 2), FPT // 2),
                          pl.ds(nbase + ch * CHUNK_N, CHUNK_N)],
                o_sems[par]).start()
        return carry

    lax.fori_loop(0, SC_CHUNKS // 2, pair_body, 0)
    wait_store(o_bufs[0], o_sems[0])
    wait_store(o_bufs[1], o_sems[1])


@jax.jit
def _sc_call(Bt, idx):
    mesh = plsc.VectorSubcoreMesh(core_axis_name="c", subcore_axis_name="s")
    f = functools.partial(
        pl.kernel,
        out_type=jax.ShapeDtypeStruct((C // 2, N_PAD), jnp.int32),
        mesh=mesh,
        compiler_params=pltpu.CompilerParams(needs_layout_passes=False),
        scratch_types=(
            [pltpu.VMEM((FPT // 2, N_PAD), jnp.int32)]
            + [pltpu.VMEM((K, CHUNK_N), jnp.int32)] * 2
            + [pltpu.VMEM((FPT // 2, CHUNK_N), jnp.int32)] * 2
            + [pltpu.SemaphoreType.DMA] * 4
        ),
    )(_sc_body)
    return f(Bt, idx)


def kernel(x, edge_index, W, b):
    x = x.astype(jnp.float32)
    ei = edge_index.astype(jnp.int32)
    A, Bp = _mm_call(x, W, b.reshape(1, C))
    ei_pad = jnp.concatenate(
        [ei, jnp.zeros((N_PAD - N_NODES, K), jnp.int32)], axis=0)
    Mt = _sc_call(Bp, ei_pad.T)
    return _elu_call(A, Mt)
